# baseline (speedup 1.0000x reference)
import functools

import jax
import jax.numpy as jnp
from jax import lax
from jax.experimental import pallas as pl
from jax.experimental.pallas import tpu as pltpu
from jax.experimental.pallas import tpu_sc as plsc

N = 10000
E = 320000
D = 128

NC = 2
NS = 16
L = 16
C = 128
KTILE = 80
NB = 8
ROWS2D = KTILE * NC * NS
NP = N + 8
ROW_SLC = 624
ROW_REM = N - NS * ROW_SLC

BLK = 1000


def _proj_body(x_ref, wt_ref, b_ref, a1w_ref, a1b_ref, a2w_ref, a2b_ref,
               h_ref, a1_ref, a2_ref):
    x = x_ref[...]
    h = jnp.dot(x, wt_ref[...], preferred_element_type=jnp.float32) + b_ref[...]
    h_ref[...] = h
    a1_ref[...] = jnp.dot(h, a1w_ref[...],
                          preferred_element_type=jnp.float32) + a1b_ref[...]
    a2_ref[...] = jnp.dot(h, a2w_ref[...],
                          preferred_element_type=jnp.float32) + a2b_ref[...]


def _project(x, wT, b2, a1wT, a1b2, a2wT, a2b2):
    return pl.pallas_call(
        _proj_body,
        grid=(N // BLK,),
        in_specs=[
            pl.BlockSpec((BLK, D), lambda i: (i, 0)),
            pl.BlockSpec((D, D), lambda i: (0, 0)),
            pl.BlockSpec((1, D), lambda i: (0, 0)),
            pl.BlockSpec((D, 1), lambda i: (0, 0)),
            pl.BlockSpec((1, 1), lambda i: (0, 0)),
            pl.BlockSpec((D, 1), lambda i: (0, 0)),
            pl.BlockSpec((1, 1), lambda i: (0, 0)),
        ],
        out_specs=[
            pl.BlockSpec((BLK, D), lambda i: (i, 0)),
            pl.BlockSpec((BLK, 1), lambda i: (i, 0)),
            pl.BlockSpec((BLK, 1), lambda i: (i, 0)),
        ],
        out_shape=[
            jax.ShapeDtypeStruct((N, D), jnp.float32),
            jax.ShapeDtypeStruct((N, 1), jnp.float32),
            jax.ShapeDtypeStruct((N, 1), jnp.float32),
        ],
    )(x, wT, b2, a1wT, a1b2, a2wT, a2b2)


def _edge_body(h_hbm, a1_hbm, a2_hbm, src_hbm, dst_hbm, z2_hbm,
               pout_hbm, pden_hbm,
               sbB0, sbB1, dbB0, dbB1, ac0, ac1, bc0, bc1, ev0, ev1,
               rw0, rw1, dbuf_v, out_sh, den_sh,
               ib0, ib1, as0, as1, gs0, gs1, ss0, ss1, ds0, ds1):
    sbB = (sbB0, sbB1)
    dbB = (dbB0, dbB1)
    ac = (ac0, ac1)
    bc = (bc0, bc1)
    ev = (ev0, ev1)
    rw = (rw0, rw1)
    ibsem = (ib0, ib1)
    asem = (as0, as1)
    gsem = (gs0, gs1)
    ssem = (ss0, ss1)
    dsem = (ds0, ds1)

    c = lax.axis_index("c")
    s = lax.axis_index("s")
    w = s * NC + c

    r0 = s * ROW_SLC

    def zbody(i, carry):
        dbuf_v[pl.ds(i * L, L)] = jnp.zeros((L,), jnp.float32)
        return carry

    lax.fori_loop(0, ROW_SLC // L, zbody, 0)
    pltpu.sync_copy(z2_hbm.at[pl.ds(r0, ROW_SLC)],
                    out_sh.at[pl.ds(r0, ROW_SLC)])
    pltpu.sync_copy(dbuf_v, den_sh.at[pl.ds(r0, ROW_SLC)])

    @pl.when(s == 0)
    def _():
        pltpu.sync_copy(z2_hbm.at[pl.ds(NS * ROW_SLC, ROW_REM)],
                        out_sh.at[pl.ds(NS * ROW_SLC, ROW_REM)])
        pltpu.sync_copy(dbuf_v.at[pl.ds(0, ROW_REM)],
                        den_sh.at[pl.ds(NS * ROW_SLC, ROW_REM)])

    plsc.subcore_barrier()

    row0 = w * KTILE

    def issue_batch(q, pb):
        pltpu.async_copy(src_hbm.at[pl.ds(row0 + q * NB, NB)],
                         sbB[pb], ibsem[pb])
        pltpu.async_copy(dst_hbm.at[pl.ds(row0 + q * NB, NB)],
                         dbB[pb], ibsem[pb])

    def wait_batch(pb):
        pltpu.make_async_copy(src_hbm.at[pl.ds(0, NB)], sbB[pb],
                              ibsem[pb]).wait()
        pltpu.make_async_copy(dst_hbm.at[pl.ds(0, NB)], dbB[pb],
                              ibsem[pb]).wait()

    def issue_avals(ba, pb, r):
        pltpu.async_copy(a1_hbm.at[sbB[pb].at[r]], ac[ba], asem[ba])
        pltpu.async_copy(a2_hbm.at[dbB[pb].at[r]], bc[ba], asem[ba])

    def wait_avals(ba, pb, r):
        pltpu.make_async_copy(a1_hbm.at[sbB[pb].at[r]], ac[ba],
                              asem[ba]).wait()
        pltpu.make_async_copy(a2_hbm.at[dbB[pb].at[r]], bc[ba],
                              asem[ba]).wait()

    def issue_gather(b2, pb, r):
        pltpu.async_copy(h_hbm.at[dbB[pb].at[r]], rw[b2], gsem[b2])

    def wait_gather(b2, pb, r):
        pltpu.make_async_copy(h_hbm.at[dbB[pb].at[r]], rw[b2],
                              gsem[b2]).wait()

    def compute_ev(b2, pb, r):
        wait_avals(b2, pb, r)
        for i in range(C // L):
            v = ac[b2][pl.ds(i * L, L)] + bc[b2][pl.ds(i * L, L)]
            v = jnp.where(v > 0, v, 0.01 * v)
            ev[b2][pl.ds(i * L, L)] = jnp.exp(v)
        pltpu.async_copy(ev[b2], den_sh.at[sbB[pb].at[r]], dsem[b2],
                         add=True)

    def drain_scatter(b2, pb, r):
        pltpu.make_async_copy(rw[b2], out_sh.at[sbB[pb].at[r]],
                              ssem[b2]).wait()
        pltpu.make_async_copy(ev[b2], den_sh.at[sbB[pb].at[r]],
                              dsem[b2]).wait()

    def scale_and_scatter(b2, pb, r):
        def body(e, carry):
            spl = plsc.load_gather(ev[b2], [jnp.full((L,), e, jnp.int32)])
            for j in range(D // L):
                rw[b2][e, pl.ds(j * L, L)] = rw[b2][e, pl.ds(j * L, L)] * spl
            return carry

        lax.fori_loop(0, C, body, 0, unroll=4)
        pltpu.async_copy(rw[b2], out_sh.at[sbB[pb].at[r]], ssem[b2],
                         add=True)

    issue_batch(0, 0)
    issue_batch(1, 1)
    wait_batch(0)
    issue_avals(0, 0, 0)
    issue_gather(0, 0, 0)

    def outer(kk, carry):
        for u in range(16):
            b2, n2 = u % 2, (u + 1) % 2
            pb, r = u // 8, u % 8
            pb1, r1 = ((u + 1) % 16) // 8, (u + 1) % 8
            if u == 6:
                wait_batch(1)
            if u == 14:
                @pl.when(kk <= 3)
                def _():
                    wait_batch(0)
            if u == 0:
                @pl.when(kk >= 1)
                def _():
                    drain_scatter(n2, 1, 7)
            else:
                drain_scatter(n2, (u - 1) // 8, (u - 1) % 8)
            if u == 2:
                @pl.when(kk >= 1)
                def _():
                    issue_batch(2 * kk + 1, 1)
            if u == 9:
                @pl.when(kk <= 3)
                def _():
                    issue_batch(2 * kk + 2, 0)
            if u == 15:
                @pl.when(kk <= 3)
                def _():
                    issue_avals(n2, 0, 0)
                    issue_gather(n2, 0, 0)
            else:
                issue_avals(n2, pb1, r1)
                issue_gather(n2, pb1, r1)
            compute_ev(b2, pb, r)
            wait_gather(b2, pb, r)
            scale_and_scatter(b2, pb, r)
        return carry

    lax.fori_loop(0, KTILE // 16, outer, 0)

    drain_scatter(1, 1, 7)
    plsc.subcore_barrier()

    pltpu.sync_copy(out_sh.at[pl.ds(r0, ROW_SLC)],
                    pout_hbm.at[c, pl.ds(r0, ROW_SLC)])
    pltpu.sync_copy(den_sh.at[pl.ds(r0, ROW_SLC)], dbuf_v)
    pltpu.sync_copy(dbuf_v,
                    pden_hbm.at[pl.ds(pl.multiple_of(c * N + r0, 8), ROW_SLC)])

    @pl.when(s == 0)
    def _():
        pltpu.sync_copy(out_sh.at[pl.ds(NS * ROW_SLC, ROW_REM)],
                        pout_hbm.at[c, pl.ds(NS * ROW_SLC, ROW_REM)])
        pltpu.sync_copy(den_sh.at[pl.ds(NS * ROW_SLC, ROW_REM)],
                        dbuf_v.at[pl.ds(0, ROW_REM)])
        pltpu.sync_copy(
            dbuf_v.at[pl.ds(0, ROW_REM)],
            pden_hbm.at[pl.ds(pl.multiple_of(c * N + NS * ROW_SLC, 8),
                              ROW_REM)])


_edge_kernel = functools.partial(
    pl.kernel,
    out_type=[
        jax.ShapeDtypeStruct((NC, N, D), jnp.float32),
        jax.ShapeDtypeStruct((NC * N,), jnp.float32),
    ],
    mesh=plsc.VectorSubcoreMesh(core_axis_name="c", subcore_axis_name="s",
                                num_cores=NC, num_subcores=NS),
    compiler_params=pltpu.CompilerParams(needs_layout_passes=False),
    scratch_types=[
        pltpu.VMEM((NB, C), jnp.int32),
        pltpu.VMEM((NB, C), jnp.int32),
        pltpu.VMEM((NB, C), jnp.int32),
        pltpu.VMEM((NB, C), jnp.int32),
        pltpu.VMEM((C,), jnp.float32),
        pltpu.VMEM((C,), jnp.float32),
        pltpu.VMEM((C,), jnp.float32),
        pltpu.VMEM((C,), jnp.float32),
        pltpu.VMEM((C,), jnp.float32),
        pltpu.VMEM((C,), jnp.float32),
        pltpu.VMEM((C, D), jnp.float32),
        pltpu.VMEM((C, D), jnp.float32),
        pltpu.VMEM((ROW_SLC,), jnp.float32),
        pltpu.VMEM_SHARED((NP, D), jnp.float32),
        pltpu.VMEM_SHARED((NP,), jnp.float32),
        pltpu.SemaphoreType.DMA,
        pltpu.SemaphoreType.DMA,
        pltpu.SemaphoreType.DMA,
        pltpu.SemaphoreType.DMA,
        pltpu.SemaphoreType.DMA,
        pltpu.SemaphoreType.DMA,
        pltpu.SemaphoreType.DMA,
        pltpu.SemaphoreType.DMA,
        pltpu.SemaphoreType.DMA,
        pltpu.SemaphoreType.DMA,
    ],
)(_edge_body)


def _combine_body(po_ref, pd_ref, out_ref):
    num = po_ref[0] + po_ref[1]
    den = pd_ref[0] + pd_ref[1]
    den = jnp.where(den == 0.0, 1.0, den)
    out_ref[...] = num / den[:, None]


def _combine(pout, pden):
    return pl.pallas_call(
        _combine_body,
        out_shape=jax.ShapeDtypeStruct((N, D), jnp.float32),
    )(pout, pden)


def kernel(features, indices, W, b, a1_w, a1_b, a2_w, a2_b):
    h, a1, a2 = _project(
        features, W.T, b.reshape(1, D),
        a1_w.reshape(1, D).T, a1_b.reshape(1, 1),
        a2_w.reshape(1, D).T, a2_b.reshape(1, 1),
    )
    src = indices[0].astype(jnp.int32)
    dst = indices[1].astype(jnp.int32)
    npad = ROWS2D * C - E
    src_p = jnp.concatenate([src, jnp.full((npad,), N, jnp.int32)])
    dst_p = jnp.concatenate([dst, jnp.zeros((npad,), jnp.int32)])
    a1p = jnp.concatenate([a1.reshape(N), jnp.zeros((L,), jnp.float32)])
    a2p = jnp.concatenate([a2.reshape(N), jnp.zeros((L,), jnp.float32)])
    z2 = jnp.zeros((N, D), jnp.float32)
    pout, pden = _edge_kernel(h, a1p, a2p, src_p.reshape(ROWS2D, C),
                              dst_p.reshape(ROWS2D, C), z2)
    return _combine(pout, pden.reshape(NC, N))


# speedup vs baseline: 1.0598x; 1.0598x over previous
import functools

import jax
import jax.numpy as jnp
from jax import lax
from jax.experimental import pallas as pl
from jax.experimental.pallas import tpu as pltpu
from jax.experimental.pallas import tpu_sc as plsc

N = 10000
E = 320000
D = 128
DE = 136

NC = 2
NS = 16
L = 16
C = 128
KTILE = 79
NROWS = 2560
NP = N + 1
ROW_SLC = 624
ROW_REM = N - NS * ROW_SLC

BLK = 1000


def _proj_body(x_ref, wt_ref, b_ref, a1w_ref, a1b_ref, a2w_ref, a2b_ref,
               h_ref, a1_ref):
    x = x_ref[...]
    h = jnp.dot(x, wt_ref[...], preferred_element_type=jnp.float32) + b_ref[...]
    a1_ref[...] = jnp.dot(h, a1w_ref[...],
                          preferred_element_type=jnp.float32) + a1b_ref[...]
    a2 = jnp.dot(h, a2w_ref[...],
                 preferred_element_type=jnp.float32) + a2b_ref[...]
    h_ref[...] = jnp.concatenate(
        [h, jnp.ones((BLK, 1), jnp.float32), a2,
         jnp.zeros((BLK, DE - D - 2), jnp.float32)], axis=1)


def _project(x, wT, b2, a1wT, a1b2, a2wT, a2b2):
    return pl.pallas_call(
        _proj_body,
        grid=(N // BLK,),
        in_specs=[
            pl.BlockSpec((BLK, D), lambda i: (i, 0)),
            pl.BlockSpec((D, D), lambda i: (0, 0)),
            pl.BlockSpec((1, D), lambda i: (0, 0)),
            pl.BlockSpec((D, 1), lambda i: (0, 0)),
            pl.BlockSpec((1, 1), lambda i: (0, 0)),
            pl.BlockSpec((D, 1), lambda i: (0, 0)),
            pl.BlockSpec((1, 1), lambda i: (0, 0)),
        ],
        out_specs=[
            pl.BlockSpec((BLK, DE), lambda i: (i, 0)),
            pl.BlockSpec((BLK, 1), lambda i: (i, 0)),
        ],
        out_shape=[
            jax.ShapeDtypeStruct((N, DE), jnp.float32),
            jax.ShapeDtypeStruct((N, 1), jnp.float32),
        ],
    )(x, wT, b2, a1wT, a1b2, a2wT, a2b2)


def _edge_body(h_hbm, a1_hbm, sd_hbm, z2_hbm, pout_hbm,
               sd0, sd1, sd2, ac0, ac1, ac2, rw0, rw1,
               out_sh,
               is0, is1, is2, as0, as1, as2, gs0, gs1, ss0, ss1):
    sd = (sd0, sd1, sd2)
    ac = (ac0, ac1, ac2)
    rw = (rw0, rw1)
    isem = (is0, is1, is2)
    asem = (as0, as1, as2)
    gsem = (gs0, gs1)
    ssem = (ss0, ss1)

    c = lax.axis_index("c")
    s = lax.axis_index("s")
    w = s * NC + c

    r0 = s * ROW_SLC
    pltpu.sync_copy(z2_hbm.at[pl.ds(r0, ROW_SLC)],
                    out_sh.at[pl.ds(r0, ROW_SLC)])

    @pl.when(s == 0)
    def _():
        pltpu.sync_copy(z2_hbm.at[pl.ds(NS * ROW_SLC, ROW_REM)],
                        out_sh.at[pl.ds(NS * ROW_SLC, ROW_REM)])

    plsc.subcore_barrier()

    def issue_idx(kq, b3):
        cid = kq * (NC * NS) + w
        pltpu.async_copy(sd_hbm.at[cid], sd[b3], isem[b3])

    def wait_idx(b3):
        pltpu.make_async_copy(sd_hbm.at[0], sd[b3], isem[b3]).wait()

    def issue_a1(b3):
        pltpu.async_copy(a1_hbm.at[sd[b3].at[0]], ac[b3], asem[b3])

    def wait_a1(b3):
        pltpu.make_async_copy(a1_hbm.at[sd[b3].at[0]], ac[b3],
                              asem[b3]).wait()

    def issue_gather(b2, b3):
        pltpu.async_copy(h_hbm.at[sd[b3].at[1]], rw[b2], gsem[b2])

    def wait_gather(b2, b3):
        pltpu.make_async_copy(h_hbm.at[sd[b3].at[1]], rw[b2],
                              gsem[b2]).wait()

    def drain_scatter(b2, b3):
        pltpu.make_async_copy(rw[b2], out_sh.at[sd[b3].at[0]],
                              ssem[b2]).wait()

    lo8 = lax.iota(jnp.int32, L) < 8

    def scale_and_scatter(b2, b3):
        def body(e, carry):
            eidx = jnp.full((L,), e, jnp.int32)
            v = (plsc.load_gather(rw[b2], [eidx, jnp.full((L,), D + 1,
                                                          jnp.int32)])
                 + plsc.load_gather(ac[b3], [eidx]))
            v = jnp.where(v > 0, v, 0.01 * v)
            ev = jnp.exp(v)
            for j in range(D // L):
                rw[b2][e, pl.ds(j * L, L)] = rw[b2][e, pl.ds(j * L, L)] * ev
            t = rw[b2][e, pl.ds(DE - L, L)]
            rw[b2][e, pl.ds(DE - L, L)] = jnp.where(lo8, t, ev)
            return carry

        lax.fori_loop(0, C, body, 0, unroll=4)
        pltpu.async_copy(rw[b2], out_sh.at[sd[b3].at[0]], ssem[b2],
                         add=True)

    issue_idx(0, 0)
    issue_idx(1, 1)
    wait_idx(0)
    issue_a1(0)
    issue_gather(0, 0)

    def outer(kk, carry):
        for u in range(6):
            k = kk * 6 + u
            b3, n3, p3 = u % 3, (u + 1) % 3, (u + 2) % 3
            b2, n2 = u % 2, (u + 1) % 2
            wait_idx(n3)
            issue_a1(n3)
            if u == 0:
                @pl.when(kk >= 1)
                def _():
                    drain_scatter(n2, p3)
            else:
                drain_scatter(n2, p3)
            issue_gather(n2, n3)
            wait_a1(b3)
            wait_gather(b2, b3)
            scale_and_scatter(b2, b3)
            if u == 5:
                @pl.when(kk <= 11)
                def _():
                    issue_idx(k + 2, p3)
            else:
                issue_idx(k + 2, p3)
        return carry

    lax.fori_loop(0, (KTILE - 1) // 6, outer, 0)

    wait_a1(0)
    wait_gather(0, 0)
    scale_and_scatter(0, 0)
    drain_scatter(1, 2)
    drain_scatter(0, 0)
    plsc.subcore_barrier()

    pltpu.sync_copy(out_sh.at[pl.ds(r0, ROW_SLC)],
                    pout_hbm.at[c, pl.ds(r0, ROW_SLC)])

    @pl.when(s == 0)
    def _():
        pltpu.sync_copy(out_sh.at[pl.ds(NS * ROW_SLC, ROW_REM)],
                        pout_hbm.at[c, pl.ds(NS * ROW_SLC, ROW_REM)])


_edge_kernel = functools.partial(
    pl.kernel,
    out_type=jax.ShapeDtypeStruct((NC, N, DE), jnp.float32),
    mesh=plsc.VectorSubcoreMesh(core_axis_name="c", subcore_axis_name="s",
                                num_cores=NC, num_subcores=NS),
    compiler_params=pltpu.CompilerParams(needs_layout_passes=False,
                                         use_tc_tiling_on_sc=False),
    scratch_types=[
        pltpu.VMEM((2, C), jnp.int32),
        pltpu.VMEM((2, C), jnp.int32),
        pltpu.VMEM((2, C), jnp.int32),
        pltpu.VMEM((C,), jnp.float32),
        pltpu.VMEM((C,), jnp.float32),
        pltpu.VMEM((C,), jnp.float32),
        pltpu.VMEM((C, DE), jnp.float32),
        pltpu.VMEM((C, DE), jnp.float32),
        pltpu.VMEM_SHARED((NP, DE), jnp.float32),
        pltpu.SemaphoreType.DMA,
        pltpu.SemaphoreType.DMA,
        pltpu.SemaphoreType.DMA,
        pltpu.SemaphoreType.DMA,
        pltpu.SemaphoreType.DMA,
        pltpu.SemaphoreType.DMA,
        pltpu.SemaphoreType.DMA,
        pltpu.SemaphoreType.DMA,
        pltpu.SemaphoreType.DMA,
        pltpu.SemaphoreType.DMA,
    ],
)(_edge_body)


def _combine_body(po_ref, out_ref):
    num = po_ref[0] + po_ref[1]
    den = num[:, D:D + 1]
    den = jnp.where(den == 0.0, 1.0, den)
    out_ref[...] = num[:, :D] / den


def _combine(pout):
    return pl.pallas_call(
        _combine_body,
        out_shape=jax.ShapeDtypeStruct((N, D), jnp.float32),
    )(pout)


def kernel(features, indices, W, b, a1_w, a1_b, a2_w, a2_b):
    h_ext, a1 = _project(
        features, W.T, b.reshape(1, D),
        a1_w.reshape(1, D).T, a1_b.reshape(1, 1),
        a2_w.reshape(1, D).T, a2_b.reshape(1, 1),
    )
    src = indices[0].astype(jnp.int32)
    dst = indices[1].astype(jnp.int32)
    npad = NROWS * C - E
    src_p = jnp.concatenate([src, jnp.full((npad,), N, jnp.int32)])
    dst_p = jnp.concatenate([dst, jnp.zeros((npad,), jnp.int32)])
    sd = jnp.stack([src_p.reshape(NROWS, C), dst_p.reshape(NROWS, C)],
                   axis=1)
    a1p = jnp.concatenate([a1.reshape(N), jnp.zeros((L,), jnp.float32)])
    z2 = jnp.zeros((N, DE), jnp.float32)
    pout = _edge_kernel(h_ext, a1p, sd, z2)
    return _combine(pout)


# speedup vs baseline: 1.4616x; 1.3791x over previous
import functools

import jax
import jax.numpy as jnp
from jax import lax
from jax.experimental import pallas as pl
from jax.experimental.pallas import tpu as pltpu
from jax.experimental.pallas import tpu_sc as plsc

N = 10000
E = 320000
D = 128

NC = 2
NS = 16
L = 16
C = 128
KTILE = 79
NROWS = 2560
NP = N + L
ROW_SLC = 624
ROW_REM = N - NS * ROW_SLC

BLK = 1000


def _proj_body(x_ref, wt_ref, b_ref, a1w_ref, a1b_ref, a2w_ref, a2b_ref,
               h_ref, a1_ref, a2_ref):
    x = x_ref[...]
    h = jnp.dot(x, wt_ref[...], preferred_element_type=jnp.float32) + b_ref[...]
    h_ref[...] = h
    a1_ref[...] = jnp.dot(h, a1w_ref[...],
                          preferred_element_type=jnp.float32) + a1b_ref[...]
    a2_ref[...] = jnp.dot(h, a2w_ref[...],
                          preferred_element_type=jnp.float32) + a2b_ref[...]


def _project(x, wT, b2, a1wT, a1b2, a2wT, a2b2):
    return pl.pallas_call(
        _proj_body,
        grid=(N // BLK,),
        in_specs=[
            pl.BlockSpec((BLK, D), lambda i: (i, 0)),
            pl.BlockSpec((D, D), lambda i: (0, 0)),
            pl.BlockSpec((1, D), lambda i: (0, 0)),
            pl.BlockSpec((D, 1), lambda i: (0, 0)),
            pl.BlockSpec((1, 1), lambda i: (0, 0)),
            pl.BlockSpec((D, 1), lambda i: (0, 0)),
            pl.BlockSpec((1, 1), lambda i: (0, 0)),
        ],
        out_specs=[
            pl.BlockSpec((BLK, D), lambda i: (i, 0)),
            pl.BlockSpec((BLK, 1), lambda i: (i, 0)),
            pl.BlockSpec((BLK, 1), lambda i: (i, 0)),
        ],
        out_shape=[
            jax.ShapeDtypeStruct((N, D), jnp.float32),
            jax.ShapeDtypeStruct((N, 1), jnp.float32),
            jax.ShapeDtypeStruct((N, 1), jnp.float32),
        ],
    )(x, wT, b2, a1wT, a1b2, a2wT, a2b2)


def _edge_body(h_hbm, a1_hbm, a2_hbm, sd_hbm, z2_hbm,
               pout_hbm, pden_hbm,
               sd0, sd1, sd2, ac0, ac1, ac2, bc0, bc1, bc2,
               ev0, ev1, ev2, rw0, rw1, dbuf_v,
               out_sh, den_sh,
               is0, is1, is2, as0, as1, as2,
               gs0, gs1, ss0, ss1, ds0, ds1, ds2):
    sd = (sd0, sd1, sd2)
    ac = (ac0, ac1, ac2)
    bc = (bc0, bc1, bc2)
    ev = (ev0, ev1, ev2)
    rw = (rw0, rw1)
    isem = (is0, is1, is2)
    asem = (as0, as1, as2)
    gsem = (gs0, gs1)
    ssem = (ss0, ss1)
    dsem = (ds0, ds1, ds2)

    c = lax.axis_index("c")
    s = lax.axis_index("s")
    w = s * NC + c

    r0 = s * ROW_SLC

    def zbody(i, carry):
        dbuf_v[pl.ds(i * L, L)] = jnp.zeros((L,), jnp.float32)
        return carry

    lax.fori_loop(0, ROW_SLC // L, zbody, 0)
    pltpu.sync_copy(z2_hbm.at[pl.ds(r0, ROW_SLC)],
                    out_sh.at[pl.ds(r0, ROW_SLC)])
    pltpu.sync_copy(dbuf_v, den_sh.at[pl.ds(r0, ROW_SLC)])

    @pl.when(s == 0)
    def _():
        pltpu.sync_copy(z2_hbm.at[pl.ds(NS * ROW_SLC, ROW_REM)],
                        out_sh.at[pl.ds(NS * ROW_SLC, ROW_REM)])
        pltpu.sync_copy(dbuf_v.at[pl.ds(0, ROW_REM)],
                        den_sh.at[pl.ds(NS * ROW_SLC, ROW_REM)])

    plsc.subcore_barrier()

    def issue_idx(kq, b3):
        cid = kq * (NC * NS) + w
        pltpu.async_copy(sd_hbm.at[cid], sd[b3], isem[b3])

    def wait_idx(b3):
        pltpu.make_async_copy(sd_hbm.at[0], sd[b3], isem[b3]).wait()

    def issue_avals(b3):
        pltpu.async_copy(a1_hbm.at[sd[b3].at[0]], ac[b3], asem[b3])
        pltpu.async_copy(a2_hbm.at[sd[b3].at[1]], bc[b3], asem[b3])

    def compute_ev(b3):
        pltpu.make_async_copy(a1_hbm.at[sd[b3].at[0]], ac[b3],
                              asem[b3]).wait()
        pltpu.make_async_copy(a2_hbm.at[sd[b3].at[1]], bc[b3],
                              asem[b3]).wait()
        for i in range(C // L):
            v = ac[b3][pl.ds(i * L, L)] + bc[b3][pl.ds(i * L, L)]
            v = jnp.where(v > 0, v, 0.01 * v)
            ev[b3][pl.ds(i * L, L)] = jnp.exp(v)
        pltpu.async_copy(ev[b3], den_sh.at[sd[b3].at[0]], dsem[b3],
                         add=True)

    def drain_scatter(b2, b3):
        pltpu.make_async_copy(rw[b2], out_sh.at[sd[b3].at[0]],
                              ssem[b2]).wait()
        pltpu.make_async_copy(ev[b3], den_sh.at[sd[b3].at[0]],
                              dsem[b3]).wait()

    def scale_and_scatter(b2, b3):
        def body(e, carry):
            spl = plsc.load_gather(ev[b3], [jnp.full((L,), e, jnp.int32)])
            for j in range(D // L):
                rw[b2][e, pl.ds(j * L, L)] = rw[b2][e, pl.ds(j * L, L)] * spl
            return carry

        lax.fori_loop(0, C, body, 0, unroll=4)
        pltpu.async_copy(rw[b2], out_sh.at[sd[b3].at[0]], ssem[b2],
                         add=True)

    issue_idx(0, 0)
    issue_idx(1, 1)
    wait_idx(0)
    issue_avals(0)
    pltpu.async_copy(h_hbm.at[sd[0].at[1]], rw[0], gsem[0])

    def outer(kk, carry):
        for u in range(6):
            k = kk * 6 + u
            b3, n3, p3 = u % 3, (u + 1) % 3, (u + 2) % 3
            b2, n2 = u % 2, (u + 1) % 2
            wait_idx(n3)
            issue_avals(n3)
            compute_ev(b3)
            if u == 0:
                @pl.when(kk >= 1)
                def _():
                    drain_scatter(n2, p3)
            else:
                drain_scatter(n2, p3)
            pltpu.async_copy(h_hbm.at[sd[n3].at[1]], rw[n2], gsem[n2])
            pltpu.make_async_copy(h_hbm.at[sd[b3].at[1]], rw[b2],
                                  gsem[b2]).wait()
            scale_and_scatter(b2, b3)
            if u == 5:
                @pl.when(kk <= 11)
                def _():
                    issue_idx(k + 2, p3)
            else:
                issue_idx(k + 2, p3)
        return carry

    lax.fori_loop(0, (KTILE - 1) // 6, outer, 0)

    compute_ev(0)
    drain_scatter(1, 2)
    pltpu.make_async_copy(h_hbm.at[sd[0].at[1]], rw[0], gsem[0]).wait()
    scale_and_scatter(0, 0)
    drain_scatter(0, 0)
    plsc.subcore_barrier()

    pltpu.sync_copy(out_sh.at[pl.ds(r0, ROW_SLC)],
                    pout_hbm.at[c, pl.ds(r0, ROW_SLC)])
    pltpu.sync_copy(den_sh.at[pl.ds(r0, ROW_SLC)], dbuf_v)
    pltpu.sync_copy(dbuf_v,
                    pden_hbm.at[pl.ds(pl.multiple_of(c * N + r0, 8), ROW_SLC)])

    @pl.when(s == 0)
    def _():
        pltpu.sync_copy(out_sh.at[pl.ds(NS * ROW_SLC, ROW_REM)],
                        pout_hbm.at[c, pl.ds(NS * ROW_SLC, ROW_REM)])
        pltpu.sync_copy(den_sh.at[pl.ds(NS * ROW_SLC, ROW_REM)],
                        dbuf_v.at[pl.ds(0, ROW_REM)])
        pltpu.sync_copy(
            dbuf_v.at[pl.ds(0, ROW_REM)],
            pden_hbm.at[pl.ds(pl.multiple_of(c * N + NS * ROW_SLC, 8),
                              ROW_REM)])


_edge_kernel = functools.partial(
    pl.kernel,
    out_type=[
        jax.ShapeDtypeStruct((NC, N, D), jnp.float32),
        jax.ShapeDtypeStruct((NC * N,), jnp.float32),
    ],
    mesh=plsc.VectorSubcoreMesh(core_axis_name="c", subcore_axis_name="s",
                                num_cores=NC, num_subcores=NS),
    compiler_params=pltpu.CompilerParams(needs_layout_passes=False),
    scratch_types=[
        pltpu.VMEM((2, C), jnp.int32),
        pltpu.VMEM((2, C), jnp.int32),
        pltpu.VMEM((2, C), jnp.int32),
        pltpu.VMEM((C,), jnp.float32),
        pltpu.VMEM((C,), jnp.float32),
        pltpu.VMEM((C,), jnp.float32),
        pltpu.VMEM((C,), jnp.float32),
        pltpu.VMEM((C,), jnp.float32),
        pltpu.VMEM((C,), jnp.float32),
        pltpu.VMEM((C,), jnp.float32),
        pltpu.VMEM((C,), jnp.float32),
        pltpu.VMEM((C,), jnp.float32),
        pltpu.VMEM((C, D), jnp.float32),
        pltpu.VMEM((C, D), jnp.float32),
        pltpu.VMEM((ROW_SLC,), jnp.float32),
        pltpu.VMEM_SHARED((NP, D), jnp.float32),
        pltpu.VMEM_SHARED((NP,), jnp.float32),
        pltpu.SemaphoreType.DMA,
        pltpu.SemaphoreType.DMA,
        pltpu.SemaphoreType.DMA,
        pltpu.SemaphoreType.DMA,
        pltpu.SemaphoreType.DMA,
        pltpu.SemaphoreType.DMA,
        pltpu.SemaphoreType.DMA,
        pltpu.SemaphoreType.DMA,
        pltpu.SemaphoreType.DMA,
        pltpu.SemaphoreType.DMA,
        pltpu.SemaphoreType.DMA,
        pltpu.SemaphoreType.DMA,
        pltpu.SemaphoreType.DMA,
    ],
)(_edge_body)


def _combine_body(po_ref, pd_ref, out_ref):
    num = po_ref[0] + po_ref[1]
    den = pd_ref[0] + pd_ref[1]
    den = jnp.where(den == 0.0, 1.0, den)
    out_ref[...] = num / den[:, None]


def _combine(pout, pden):
    return pl.pallas_call(
        _combine_body,
        out_shape=jax.ShapeDtypeStruct((N, D), jnp.float32),
    )(pout, pden)


def kernel(features, indices, W, b, a1_w, a1_b, a2_w, a2_b):
    h, a1, a2 = _project(
        features, W.T, b.reshape(1, D),
        a1_w.reshape(1, D).T, a1_b.reshape(1, 1),
        a2_w.reshape(1, D).T, a2_b.reshape(1, 1),
    )
    src = indices[0].astype(jnp.int32)
    dst = indices[1].astype(jnp.int32)
    npad = NROWS * C - E
    src_p = jnp.concatenate([src, jnp.full((npad,), N, jnp.int32)])
    dst_p = jnp.concatenate([dst, jnp.zeros((npad,), jnp.int32)])
    sd = jnp.stack([src_p.reshape(NROWS, C), dst_p.reshape(NROWS, C)],
                   axis=1)
    a1p = jnp.concatenate([a1.reshape(N), jnp.zeros((L,), jnp.float32)])
    a2p = jnp.concatenate([a2.reshape(N), jnp.zeros((L,), jnp.float32)])
    z2 = jnp.zeros((N, D), jnp.float32)
    pout, pden = _edge_kernel(h, a1p, a2p, sd, z2)
    return _combine(pout, pden.reshape(NC, N))
